# Initial kernel scaffold; baseline (speedup 1.0000x reference)
import jax
import jax.numpy as jnp
from jax.experimental import pallas as pl


def kernel(inputs, w, b, indices):
    raise NotImplementedError("write your pallas kernel here")


# speedup vs baseline: 34.3513x; 34.3513x over previous
import functools

import jax
import jax.numpy as jnp
from jax import lax
from jax.experimental import pallas as pl
from jax.experimental.pallas import tpu as pltpu
from jax.experimental.pallas import tpu_sc as plsc

_LANES = 16
_N_WORKERS = 32


def _build_wt_sc(units, input_dim, mc):
    units_per_w = units // _N_WORKERS
    row_len = units_per_w * input_dim
    flat_len = units_per_w * 8 * mc
    n_chunks = flat_len // _LANES
    zero_chunks = row_len // _LANES

    mesh = plsc.VectorSubcoreMesh(core_axis_name="c", subcore_axis_name="s")

    @functools.partial(
        pl.kernel,
        mesh=mesh,
        out_type=jax.ShapeDtypeStruct((_N_WORKERS, row_len), jnp.float32),
        scratch_types=[
            pltpu.VMEM((flat_len,), jnp.int32),
            pltpu.VMEM((_LANES,), jnp.float32),
            pltpu.VMEM((row_len,), jnp.float32),
        ],
        compiler_params=pltpu.CompilerParams(needs_layout_passes=False),
    )
    def build(ind_hbm, w_hbm, wt_hbm, ind_v, w_v, rows_v):
        wid = lax.axis_index("s") * 2 + lax.axis_index("c")
        pltpu.sync_copy(ind_hbm.at[wid], ind_v)
        pltpu.sync_copy(w_hbm, w_v)

        def zero_body(i, _):
            rows_v[pl.ds(i * _LANES, _LANES)] = jnp.zeros(
                (_LANES,), jnp.float32)
            return 0
        lax.fori_loop(0, zero_chunks, zero_body, 0)

        lane_iota = lax.iota(jnp.int32, _LANES)

        def full(v):
            return jnp.full((_LANES,), v, jnp.int32)

        def body(i, _):
            idx = ind_v[pl.ds(i * _LANES, _LANES)]
            flat = full(i * _LANES) + lane_iota
            row = lax.div(flat, full(8 * mc))
            k = lax.rem(lax.div(flat, full(mc)), full(8))
            val = lax.gather(
                w_v[...], k[:, None],
                dimension_numbers=lax.GatherDimensionNumbers(
                    offset_dims=(), collapsed_slice_dims=(0,),
                    start_index_map=(0,)),
                slice_sizes=(1,),
                mode=lax.GatherScatterMode.PROMISE_IN_BOUNDS)
            mask = lax.gt(idx, full(0))
            pos = lax.add(lax.mul(row, full(input_dim)),
                          lax.sub(idx, full(1)))
            plsc.store_scatter(rows_v, [pos], val, mask=mask)
            return 0
        lax.fori_loop(0, n_chunks, body, 0)

        pltpu.sync_copy(rows_v, wt_hbm.at[wid])

    return build


def _matmul_tc(x, wt, b2):
    def body(x_ref, wt_ref, b_ref, o_ref):
        o_ref[...] = lax.dot_general(
            x_ref[...], wt_ref[...],
            dimension_numbers=(((1,), (1,)), ((), ())),
            preferred_element_type=jnp.float32,
        ) + b_ref[...]

    return pl.pallas_call(
        body,
        out_shape=jax.ShapeDtypeStruct((x.shape[0], wt.shape[0]),
                                       jnp.float32),
    )(x, wt, b2)


def kernel(inputs, w, b, indices):
    units, n_w, mc = indices.shape
    input_dim = inputs.shape[1]
    ind2 = indices.astype(jnp.int32).reshape(_N_WORKERS, -1)
    w16 = jnp.pad(w.astype(jnp.float32), (0, _LANES - n_w))
    wt_rows = _build_wt_sc(units, input_dim, mc)(ind2, w16)
    wt = wt_rows.reshape(units, input_dim)
    return _matmul_tc(inputs, wt, b.reshape(1, units))


# speedup vs baseline: 41.9179x; 1.2203x over previous
import functools

import jax
import jax.numpy as jnp
from jax import lax
from jax.experimental import pallas as pl
from jax.experimental.pallas import tpu as pltpu
from jax.experimental.pallas import tpu_sc as plsc

_LANES = 16
_N_WORKERS = 32


def _chunk_starts(mc):
    starts = list(range(0, mc - _LANES + 1, _LANES))
    if mc % _LANES:
        starts.append(mc - _LANES)
    return starts


def _build_wt_sc(units, input_dim, n_w, mc):
    units_per_w = units // _N_WORKERS
    row_len = units_per_w * input_dim
    zero_chunks = row_len // _LANES
    starts = _chunk_starts(mc)

    mesh = plsc.VectorSubcoreMesh(core_axis_name="c", subcore_axis_name="s")

    @functools.partial(
        pl.kernel,
        mesh=mesh,
        out_type=jax.ShapeDtypeStruct((units, input_dim), jnp.float32),
        scratch_types=[
            pltpu.VMEM((units_per_w, n_w, mc), jnp.int32),
            pltpu.VMEM((_LANES,), jnp.float32),
            pltpu.VMEM((row_len,), jnp.float32),
            pltpu.SemaphoreType.DMA,
        ],
        compiler_params=pltpu.CompilerParams(needs_layout_passes=False,
                                             use_tc_tiling_on_sc=True),
    )
    def build(ind_hbm, w_hbm, wt_hbm, ind_v, w_v, rows_v, sem):
        wid = lax.axis_index("s") * 2 + lax.axis_index("c")
        ubase = lax.mul(wid, units_per_w)
        cp_ind = pltpu.make_async_copy(
            ind_hbm.at[pl.ds(ubase, units_per_w)], ind_v, sem)
        cp_ind.start()
        pltpu.sync_copy(w_hbm, w_v.at[pl.ds(0, n_w)])

        @plsc.parallel_loop(0, zero_chunks, unroll=8)
        def zero_body(i):
            rows_v[pl.ds(lax.mul(i, _LANES), _LANES)] = jnp.zeros(
                (_LANES,), jnp.float32)

        cp_ind.wait()
        wv = w_v[...]

        def full(v):
            return jnp.full((_LANES,), v, jnp.int32)

        n_ci = len(starts)

        @plsc.parallel_loop(0, units_per_w * n_w * n_ci, unroll=4)
        def body(t):
            ci = lax.rem(t, n_ci)
            tk = lax.div(t, n_ci)
            k = lax.rem(tk, n_w)
            j = lax.div(tk, n_w)
            c = lax.min(lax.mul(ci, _LANES), mc - _LANES)
            idx = ind_v[j, k, pl.ds(c, _LANES)]
            val = lax.gather(
                wv, jnp.full((_LANES,), k, jnp.int32)[:, None],
                dimension_numbers=lax.GatherDimensionNumbers(
                    offset_dims=(), collapsed_slice_dims=(0,),
                    start_index_map=(0,)),
                slice_sizes=(1,),
                mode=lax.GatherScatterMode.PROMISE_IN_BOUNDS)
            mask = lax.gt(idx, full(0))
            pos = lax.add(idx, jnp.full((_LANES,), lax.sub(
                lax.mul(j, input_dim), 1), jnp.int32))
            plsc.store_scatter(rows_v, [pos], val, mask=mask)

        cps = [
            pltpu.make_async_copy(
                rows_v.at[pl.ds(j * input_dim, input_dim)],
                wt_hbm.at[lax.add(ubase, j)], sem)
            for j in range(units_per_w)
        ]
        for cp in cps:
            cp.start()
        for cp in cps:
            cp.wait()

    return build


def _matmul_tc(x, wt, b2):
    def body(x_ref, wt_ref, b_ref, o_ref):
        o_ref[...] = lax.dot_general(
            x_ref[...], wt_ref[...],
            dimension_numbers=(((1,), (1,)), ((), ())),
            preferred_element_type=jnp.float32,
        ) + b_ref[...]

    return pl.pallas_call(
        body,
        out_shape=jax.ShapeDtypeStruct((x.shape[0], wt.shape[0]),
                                       jnp.float32),
    )(x, wt, b2)


def kernel(inputs, w, b, indices):
    units, n_w, mc = indices.shape
    input_dim = inputs.shape[1]
    wt = _build_wt_sc(units, input_dim, n_w, mc)(
        indices.astype(jnp.int32), w.astype(jnp.float32))
    return _matmul_tc(inputs, wt, b.reshape(1, units))
